# baseline (speedup 1.0000x reference)
import functools

import jax
import jax.numpy as jnp
from jax import lax
from jax.experimental import pallas as pl
from jax.experimental.pallas import tpu as pltpu
from jax.experimental.pallas import tpu_sc as plsc

N = 10000
E = 320000
D = 128
G = 64

NC = 2
NS = 16
NW = NC * NS
EW = E // NW
C = 125
NCHUNK = EW // C
CE = 125
NCHUNKE = EW // CE
SR = 80
N_PAD = 10240
RPS = N_PAD // NS
NSTG = RPS // SR

PHASES = ((0, 52), (52, NCHUNKE - 52))

_mesh = plsc.VectorSubcoreMesh(core_axis_name="c", subcore_axis_name="s")

OUT_SC = jax.ShapeDtypeStruct((NC, N_PAD, D), jnp.float32)


def _fill(buf, nrows, val):
    v16 = jnp.full((16,), val, jnp.float32)

    def f(i, carry):
        for k in range(D // 16):
            buf[i, pl.ds(k * 16, 16)] = v16
        return carry

    lax.fori_loop(0, nrows, f, 0)


@functools.partial(
    pl.kernel,
    out_type=OUT_SC,
    mesh=_mesh,
    scratch_types=[
        pltpu.VMEM((NCHUNK, C), jnp.int32),
        pltpu.VMEM((C, D), jnp.float32),
        pltpu.VMEM((C, D), jnp.float32),
        pltpu.VMEM_SHARED((N_PAD, D), jnp.float32),
    ],
)
def _deg_kernel(dst_hbm, out_hbm, dst_v, ones_v, stage, acc):
    c = lax.axis_index("c")
    s = lax.axis_index("s")
    w = s * NC + c
    base = s * RPS

    _fill(stage, C, 0.0)
    _fill(ones_v, C, 1.0)
    for k in range(NSTG):
        pltpu.sync_copy(stage.at[pl.ds(0, SR)],
                        acc.at[pl.ds(base + k * SR, SR)])
    pltpu.sync_copy(dst_hbm.at[w], dst_v)
    plsc.subcore_barrier()

    def chunk(j, carry):
        pltpu.sync_copy(ones_v, acc.at[dst_v.at[j]], add=True)
        return carry

    lax.fori_loop(0, NCHUNK, chunk, 0)

    plsc.subcore_barrier()
    for k in range(NSTG):
        pltpu.sync_copy(acc.at[pl.ds(base + k * SR, SR)],
                        stage.at[pl.ds(0, SR)])
        pltpu.sync_copy(stage.at[pl.ds(0, SR)],
                        out_hbm.at[c].at[pl.ds(base + k * SR, SR)])


@functools.partial(
    pl.kernel,
    out_type=OUT_SC,
    mesh=_mesh,
    scratch_types=[
        pltpu.VMEM((2 * 52, CE), jnp.int32),
        pltpu.VMEM((CE, D), jnp.float32),
        pltpu.VMEM((CE, D), jnp.float32),
        pltpu.SemaphoreType.DMA,
        pltpu.SemaphoreType.DMA,
        pltpu.VMEM_SHARED((N_PAD, D), jnp.float32),
    ],
)
def _edge_kernel(hp_hbm, idx_hbm, out_hbm, idx_v, rows0, rows1,
                 sem0, sem1, acc):
    c = lax.axis_index("c")
    s = lax.axis_index("s")
    w = s * NC + c
    base = s * RPS

    @pl.when(c == 0)
    def _():
        for k in range(NSTG):
            pltpu.sync_copy(hp_hbm.at[pl.ds(base + k * SR, SR)],
                            rows0.at[pl.ds(0, SR)])
            pltpu.sync_copy(rows0.at[pl.ds(0, SR)],
                            acc.at[pl.ds(base + k * SR, SR)])

    @pl.when(c != 0)
    def _():
        _fill(rows0, SR, 0.0)
        for k in range(NSTG):
            pltpu.sync_copy(rows0.at[pl.ds(0, SR)],
                            acc.at[pl.ds(base + k * SR, SR)])

    plsc.subcore_barrier()

    for start, npair in PHASES:
        pltpu.sync_copy(idx_hbm.at[w].at[pl.ds(2 * start, 2 * npair)],
                        idx_v.at[pl.ds(0, 2 * npair)])
        pltpu.async_copy(hp_hbm.at[idx_v.at[0]], rows0, sem0)

        def pair(p, carry):
            even = (p % 2) == 0

            @pl.when(even)
            def _():
                pltpu.make_async_copy(hp_hbm.at[idx_v.at[0]], rows0,
                                      sem0).wait()

                @pl.when(p + 1 < npair)
                def _():
                    pltpu.async_copy(hp_hbm.at[idx_v.at[2 * (p + 1)]],
                                     rows1, sem1)
                pltpu.sync_copy(rows0, acc.at[idx_v.at[2 * p + 1]], add=True)

            @pl.when(jnp.logical_not(even))
            def _():
                pltpu.make_async_copy(hp_hbm.at[idx_v.at[0]], rows1,
                                      sem1).wait()

                @pl.when(p + 1 < npair)
                def _():
                    pltpu.async_copy(hp_hbm.at[idx_v.at[2 * (p + 1)]],
                                     rows0, sem0)
                pltpu.sync_copy(rows1, acc.at[idx_v.at[2 * p + 1]], add=True)

            return carry

        lax.fori_loop(0, npair, pair, 0)

    plsc.subcore_barrier()
    for k in range(NSTG):
        pltpu.sync_copy(acc.at[pl.ds(base + k * SR, SR)],
                        rows0.at[pl.ds(0, SR)])
        pltpu.sync_copy(rows0.at[pl.ds(0, SR)],
                        out_hbm.at[c].at[pl.ds(base + k * SR, SR)])


R_BLK = 1000
N_BLKS = N // R_BLK


def _mm_scale_body(degp_ref, x_ref, w_ref, hp_ref, dinv_ref):
    deg = 1.0 + degp_ref[0, :, 0] + degp_ref[1, :, 0]
    dinv = lax.rsqrt(deg)[:, None]
    h = jnp.dot(x_ref[...], w_ref[...], preferred_element_type=jnp.float32)
    hp_ref[...] = h * dinv
    dinv_ref[...] = dinv


def _elu(t):
    return jnp.where(t > 0.0, t, jnp.exp(jnp.minimum(t, 0.0)) - 1.0)


def _mid_body(p_ref, dinv_ref, b_ref, w_ref, hp2_ref):
    dinv = dinv_ref[...]
    t = dinv * (p_ref[0] + p_ref[1]) + b_ref[...]
    h = _elu(t)
    h2 = jnp.dot(h, w_ref[...], preferred_element_type=jnp.float32)
    hp2_ref[...] = h2 * dinv


def _final_body(p_ref, dinv_ref, b_ref, batch_ref, wl_ref, bl_ref,
                out_ref, pool_acc, cnt_acc):
    i = pl.program_id(0)

    @pl.when(i == 0)
    def _():
        pool_acc[...] = jnp.zeros_like(pool_acc)
        cnt_acc[...] = jnp.zeros_like(cnt_acc)

    dinv = dinv_ref[...]
    t = dinv * (p_ref[0] + p_ref[1]) + b_ref[...]
    h = _elu(t)
    b = batch_ref[...]
    onehot = (b == lax.broadcasted_iota(jnp.int32, (R_BLK, G), 1)
              ).astype(jnp.float32)
    dn = (((0,), (0,)), ((), ()))
    pool_acc[...] += lax.dot_general(onehot, h, dn,
                                     preferred_element_type=jnp.float32)
    cnt_acc[...] += lax.dot_general(onehot, jnp.ones((R_BLK, D), jnp.float32),
                                    dn, preferred_element_type=jnp.float32)

    @pl.when(i == pl.num_programs(0) - 1)
    def _():
        pooled = pool_acc[...] / jnp.maximum(cnt_acc[...], 1.0)
        out_ref[...] = (jnp.dot(pooled, wl_ref[...],
                                preferred_element_type=jnp.float32)
                        + bl_ref[...])


_mm_scale = pl.pallas_call(
    _mm_scale_body,
    grid=(N_BLKS,),
    in_specs=[
        pl.BlockSpec((NC, R_BLK, D), lambda i: (0, i, 0)),
        pl.BlockSpec((R_BLK, D), lambda i: (i, 0)),
        pl.BlockSpec((D, D), lambda i: (0, 0)),
    ],
    out_specs=[
        pl.BlockSpec((R_BLK, D), lambda i: (i, 0)),
        pl.BlockSpec((R_BLK, 1), lambda i: (i, 0)),
    ],
    out_shape=[
        jax.ShapeDtypeStruct((N_PAD, D), jnp.float32),
        jax.ShapeDtypeStruct((N, 1), jnp.float32),
    ],
)

_mid = pl.pallas_call(
    _mid_body,
    grid=(N_BLKS,),
    in_specs=[
        pl.BlockSpec((NC, R_BLK, D), lambda i: (0, i, 0)),
        pl.BlockSpec((R_BLK, 1), lambda i: (i, 0)),
        pl.BlockSpec((1, D), lambda i: (0, 0)),
        pl.BlockSpec((D, D), lambda i: (0, 0)),
    ],
    out_specs=pl.BlockSpec((R_BLK, D), lambda i: (i, 0)),
    out_shape=jax.ShapeDtypeStruct((N_PAD, D), jnp.float32),
)

_final = pl.pallas_call(
    _final_body,
    grid=(N_BLKS,),
    in_specs=[
        pl.BlockSpec((NC, R_BLK, D), lambda i: (0, i, 0)),
        pl.BlockSpec((R_BLK, 1), lambda i: (i, 0)),
        pl.BlockSpec((1, D), lambda i: (0, 0)),
        pl.BlockSpec((R_BLK, 1), lambda i: (i, 0)),
        pl.BlockSpec((D, 1), lambda i: (0, 0)),
        pl.BlockSpec((1, 1), lambda i: (0, 0)),
    ],
    out_specs=pl.BlockSpec((G, 1), lambda i: (0, 0)),
    out_shape=jax.ShapeDtypeStruct((G, 1), jnp.float32),
    scratch_shapes=[
        pltpu.VMEM((G, D), jnp.float32),
        pltpu.VMEM((G, D), jnp.float32),
    ],
)


def kernel(x, edge_index, batch, W1, b1, W2, b2, Wl, bl):
    dst3 = edge_index[1].reshape(NW, NCHUNK, C)
    src3e = edge_index[0].reshape(NW, NCHUNKE, CE)
    dst3e = edge_index[1].reshape(NW, NCHUNKE, CE)
    idx3 = jnp.stack([src3e, dst3e], axis=2).reshape(NW, 2 * NCHUNKE, CE)

    degp = _deg_kernel(dst3)
    hp1, dinv = _mm_scale(degp, x, W1)

    p1 = _edge_kernel(hp1, idx3)
    hp2 = _mid(p1, dinv, b1.reshape(1, D), W2)
    p2 = _edge_kernel(hp2, idx3)

    batch_i = batch.reshape(N, 1)
    return _final(p2, dinv, b2.reshape(1, D), batch_i,
                  Wl, bl.reshape(1, 1))


# speedup vs baseline: 1.0313x; 1.0313x over previous
import functools

import jax
import jax.numpy as jnp
from jax import lax
from jax.experimental import pallas as pl
from jax.experimental.pallas import tpu as pltpu
from jax.experimental.pallas import tpu_sc as plsc

N = 10000
E = 320000
D = 128
G = 64

NC = 2
NS = 16
NW = NC * NS
EW = E // NW
C = 125
NCHUNK = EW // C
CE = 125
NCHUNKE = EW // CE
SR = 80
N_PAD = 10240
RPS = N_PAD // NS
NSTG = RPS // SR

PHASES = ((0, 52), (52, NCHUNKE - 52))

_mesh = plsc.VectorSubcoreMesh(core_axis_name="c", subcore_axis_name="s")

OUT_SC = jax.ShapeDtypeStruct((NC, N_PAD, D), jnp.float32)


def _fill(buf, nrows, val):
    v16 = jnp.full((16,), val, jnp.float32)

    def f(i, carry):
        for k in range(D // 16):
            buf[i, pl.ds(k * 16, 16)] = v16
        return carry

    lax.fori_loop(0, nrows, f, 0)


@functools.partial(
    pl.kernel,
    out_type=OUT_SC,
    mesh=_mesh,
    scratch_types=[
        pltpu.VMEM((NCHUNK, C), jnp.int32),
        pltpu.VMEM((C, D), jnp.float32),
        pltpu.VMEM((C, D), jnp.float32),
        pltpu.SemaphoreType.DMA,
        pltpu.SemaphoreType.DMA,
        pltpu.VMEM_SHARED((N_PAD, D), jnp.float32),
    ],
)
def _deg_kernel(dst_hbm, out_hbm, dst_v, ones_v, stage, sem0, sem1, acc):
    c = lax.axis_index("c")
    s = lax.axis_index("s")
    w = s * NC + c
    base = s * RPS

    _fill(stage, C, 0.0)
    _fill(ones_v, C, 1.0)
    for k in range(NSTG):
        pltpu.async_copy(stage.at[pl.ds(0, SR)],
                         acc.at[pl.ds(base + k * SR, SR)], sem0)
    pltpu.sync_copy(dst_hbm.at[w], dst_v)
    for k in range(NSTG):
        pltpu.make_async_copy(stage.at[pl.ds(0, SR)],
                              acc.at[pl.ds(base, SR)], sem0).wait()
    plsc.subcore_barrier()

    def chunk(j, carry):
        pltpu.sync_copy(ones_v, acc.at[dst_v.at[j]], add=True)
        return carry

    lax.fori_loop(0, NCHUNK, chunk, 0)

    plsc.subcore_barrier()
    robufs = ((stage, sem0), (ones_v, sem1))
    for k in range(NSTG):
        buf, sem = robufs[k % 2]
        if k >= 2:
            pltpu.make_async_copy(
                buf.at[pl.ds(0, SR)],
                out_hbm.at[c].at[pl.ds(base + (k - 2) * SR, SR)], sem).wait()
        pltpu.sync_copy(acc.at[pl.ds(base + k * SR, SR)], buf.at[pl.ds(0, SR)])
        pltpu.async_copy(buf.at[pl.ds(0, SR)],
                         out_hbm.at[c].at[pl.ds(base + k * SR, SR)], sem)
    for k in (NSTG - 2, NSTG - 1):
        buf, sem = robufs[k % 2]
        pltpu.make_async_copy(buf.at[pl.ds(0, SR)],
                              out_hbm.at[c].at[pl.ds(base + k * SR, SR)],
                              sem).wait()


@functools.partial(
    pl.kernel,
    out_type=OUT_SC,
    mesh=_mesh,
    scratch_types=[
        pltpu.VMEM((2 * 52, CE), jnp.int32),
        pltpu.VMEM((CE, D), jnp.float32),
        pltpu.VMEM((CE, D), jnp.float32),
        pltpu.SemaphoreType.DMA,
        pltpu.SemaphoreType.DMA,
        pltpu.VMEM_SHARED((N_PAD, D), jnp.float32),
    ],
)
def _edge_kernel(hp_hbm, idx_hbm, out_hbm, idx_v, rows0, rows1,
                 sem0, sem1, acc):
    c = lax.axis_index("c")
    s = lax.axis_index("s")
    w = s * NC + c
    base = s * RPS

    ebufs = ((rows0, sem0), (rows1, sem1))

    @pl.when(c == 0)
    def _():
        pltpu.async_copy(hp_hbm.at[pl.ds(base, SR)],
                         rows0.at[pl.ds(0, SR)], sem0)
        for k in range(NSTG):
            buf, sem = ebufs[k % 2]
            pltpu.make_async_copy(hp_hbm.at[pl.ds(base + k * SR, SR)],
                                  buf.at[pl.ds(0, SR)], sem).wait()
            if k + 1 < NSTG:
                nbuf, nsem = ebufs[(k + 1) % 2]
                pltpu.async_copy(hp_hbm.at[pl.ds(base + (k + 1) * SR, SR)],
                                 nbuf.at[pl.ds(0, SR)], nsem)
            pltpu.sync_copy(buf.at[pl.ds(0, SR)],
                            acc.at[pl.ds(base + k * SR, SR)])

    @pl.when(c != 0)
    def _():
        _fill(rows0, SR, 0.0)
        for k in range(NSTG):
            pltpu.async_copy(rows0.at[pl.ds(0, SR)],
                             acc.at[pl.ds(base + k * SR, SR)], sem0)
        for k in range(NSTG):
            pltpu.make_async_copy(rows0.at[pl.ds(0, SR)],
                                  acc.at[pl.ds(base, SR)], sem0).wait()

    plsc.subcore_barrier()

    for start, npair in PHASES:
        pltpu.sync_copy(idx_hbm.at[w].at[pl.ds(2 * start, 2 * npair)],
                        idx_v.at[pl.ds(0, 2 * npair)])
        pltpu.async_copy(hp_hbm.at[idx_v.at[0]], rows0, sem0)

        def pair(p, carry):
            even = (p % 2) == 0

            @pl.when(even)
            def _():
                pltpu.make_async_copy(hp_hbm.at[idx_v.at[0]], rows0,
                                      sem0).wait()

                @pl.when(p + 1 < npair)
                def _():
                    pltpu.async_copy(hp_hbm.at[idx_v.at[2 * (p + 1)]],
                                     rows1, sem1)
                pltpu.sync_copy(rows0, acc.at[idx_v.at[2 * p + 1]], add=True)

            @pl.when(jnp.logical_not(even))
            def _():
                pltpu.make_async_copy(hp_hbm.at[idx_v.at[0]], rows1,
                                      sem1).wait()

                @pl.when(p + 1 < npair)
                def _():
                    pltpu.async_copy(hp_hbm.at[idx_v.at[2 * (p + 1)]],
                                     rows0, sem0)
                pltpu.sync_copy(rows1, acc.at[idx_v.at[2 * p + 1]], add=True)

            return carry

        lax.fori_loop(0, npair, pair, 0)

    plsc.subcore_barrier()
    for k in range(NSTG):
        buf, sem = ebufs[k % 2]
        if k >= 2:
            pltpu.make_async_copy(
                buf.at[pl.ds(0, SR)],
                out_hbm.at[c].at[pl.ds(base + (k - 2) * SR, SR)], sem).wait()
        pltpu.sync_copy(acc.at[pl.ds(base + k * SR, SR)], buf.at[pl.ds(0, SR)])
        pltpu.async_copy(buf.at[pl.ds(0, SR)],
                         out_hbm.at[c].at[pl.ds(base + k * SR, SR)], sem)
    for k in (NSTG - 2, NSTG - 1):
        buf, sem = ebufs[k % 2]
        pltpu.make_async_copy(buf.at[pl.ds(0, SR)],
                              out_hbm.at[c].at[pl.ds(base + k * SR, SR)],
                              sem).wait()


R_BLK = 1000
N_BLKS = N // R_BLK


def _mm_scale_body(degp_ref, x_ref, w_ref, hp_ref, dinv_ref):
    deg = 1.0 + degp_ref[0, :, 0] + degp_ref[1, :, 0]
    dinv = lax.rsqrt(deg)[:, None]
    h = jnp.dot(x_ref[...], w_ref[...], preferred_element_type=jnp.float32)
    hp_ref[...] = h * dinv
    dinv_ref[...] = dinv


def _elu(t):
    return jnp.where(t > 0.0, t, jnp.exp(jnp.minimum(t, 0.0)) - 1.0)


def _mid_body(p_ref, dinv_ref, b_ref, w_ref, hp2_ref):
    dinv = dinv_ref[...]
    t = dinv * (p_ref[0] + p_ref[1]) + b_ref[...]
    h = _elu(t)
    h2 = jnp.dot(h, w_ref[...], preferred_element_type=jnp.float32)
    hp2_ref[...] = h2 * dinv


def _final_body(p_ref, dinv_ref, b_ref, batch_ref, wl_ref, bl_ref,
                out_ref, pool_acc, cnt_acc):
    i = pl.program_id(0)

    @pl.when(i == 0)
    def _():
        pool_acc[...] = jnp.zeros_like(pool_acc)
        cnt_acc[...] = jnp.zeros_like(cnt_acc)

    dinv = dinv_ref[...]
    t = dinv * (p_ref[0] + p_ref[1]) + b_ref[...]
    h = _elu(t)
    b = batch_ref[...]
    onehot = (b == lax.broadcasted_iota(jnp.int32, (R_BLK, G), 1)
              ).astype(jnp.float32)
    dn = (((0,), (0,)), ((), ()))
    pool_acc[...] += lax.dot_general(onehot, h, dn,
                                     preferred_element_type=jnp.float32)
    cnt_acc[...] += lax.dot_general(onehot, jnp.ones((R_BLK, D), jnp.float32),
                                    dn, preferred_element_type=jnp.float32)

    @pl.when(i == pl.num_programs(0) - 1)
    def _():
        pooled = pool_acc[...] / jnp.maximum(cnt_acc[...], 1.0)
        out_ref[...] = (jnp.dot(pooled, wl_ref[...],
                                preferred_element_type=jnp.float32)
                        + bl_ref[...])


_mm_scale = pl.pallas_call(
    _mm_scale_body,
    grid=(N_BLKS,),
    in_specs=[
        pl.BlockSpec((NC, R_BLK, D), lambda i: (0, i, 0)),
        pl.BlockSpec((R_BLK, D), lambda i: (i, 0)),
        pl.BlockSpec((D, D), lambda i: (0, 0)),
    ],
    out_specs=[
        pl.BlockSpec((R_BLK, D), lambda i: (i, 0)),
        pl.BlockSpec((R_BLK, 1), lambda i: (i, 0)),
    ],
    out_shape=[
        jax.ShapeDtypeStruct((N_PAD, D), jnp.float32),
        jax.ShapeDtypeStruct((N, 1), jnp.float32),
    ],
)

_mid = pl.pallas_call(
    _mid_body,
    grid=(N_BLKS,),
    in_specs=[
        pl.BlockSpec((NC, R_BLK, D), lambda i: (0, i, 0)),
        pl.BlockSpec((R_BLK, 1), lambda i: (i, 0)),
        pl.BlockSpec((1, D), lambda i: (0, 0)),
        pl.BlockSpec((D, D), lambda i: (0, 0)),
    ],
    out_specs=pl.BlockSpec((R_BLK, D), lambda i: (i, 0)),
    out_shape=jax.ShapeDtypeStruct((N_PAD, D), jnp.float32),
)

_final = pl.pallas_call(
    _final_body,
    grid=(N_BLKS,),
    in_specs=[
        pl.BlockSpec((NC, R_BLK, D), lambda i: (0, i, 0)),
        pl.BlockSpec((R_BLK, 1), lambda i: (i, 0)),
        pl.BlockSpec((1, D), lambda i: (0, 0)),
        pl.BlockSpec((R_BLK, 1), lambda i: (i, 0)),
        pl.BlockSpec((D, 1), lambda i: (0, 0)),
        pl.BlockSpec((1, 1), lambda i: (0, 0)),
    ],
    out_specs=pl.BlockSpec((G, 1), lambda i: (0, 0)),
    out_shape=jax.ShapeDtypeStruct((G, 1), jnp.float32),
    scratch_shapes=[
        pltpu.VMEM((G, D), jnp.float32),
        pltpu.VMEM((G, D), jnp.float32),
    ],
)


def kernel(x, edge_index, batch, W1, b1, W2, b2, Wl, bl):
    dst3 = edge_index[1].reshape(NW, NCHUNK, C)
    src3e = edge_index[0].reshape(NW, NCHUNKE, CE)
    dst3e = edge_index[1].reshape(NW, NCHUNKE, CE)
    idx3 = jnp.stack([src3e, dst3e], axis=2).reshape(NW, 2 * NCHUNKE, CE)

    degp = _deg_kernel(dst3)
    hp1, dinv = _mm_scale(degp, x, W1)

    p1 = _edge_kernel(hp1, idx3)
    hp2 = _mid(p1, dinv, b1.reshape(1, D), W2)
    p2 = _edge_kernel(hp2, idx3)

    batch_i = batch.reshape(N, 1)
    return _final(p2, dinv, b2.reshape(1, D), batch_i,
                  Wl, bl.reshape(1, 1))
